# baseline (speedup 1.0000x reference)
import functools

import jax
import jax.numpy as jnp
from jax import lax
from jax.experimental import pallas as pl
from jax.experimental.pallas import tpu as pltpu
from jax.experimental.pallas import tpu_sc as plsc

N = 50000
M = 16
IN_D = 128
OC = 8
KD = 8
D = OC * KD
ROUT_IT = 2

N_SLICES = 5
NODES_SL = N // N_SLICES
EDGES_SL = NODES_SL * M

BN = 1000
NC, NS = 2, 16
NW = NC * NS
PER_W = EDGES_SL // NW
CHUNK = 1000
N_CHUNKS = PER_W // CHUNK


def _selector():
    d_idx = lax.broadcasted_iota(jnp.int32, (D, OC), 0)
    c_idx = lax.broadcasted_iota(jnp.int32, (D, OC), 1)
    return jnp.where(d_idx // KD == c_idx, 1.0, 0.0).astype(jnp.float32)


def _selector8():
    d_idx = lax.broadcasted_iota(jnp.int32, (8 * D, 8 * OC), 0)
    c_idx = lax.broadcasted_iota(jnp.int32, (8 * D, 8 * OC), 1)
    return jnp.where(d_idx // KD == c_idx, 1.0, 0.0).astype(jnp.float32)


def _blocksum8():
    a_idx = lax.broadcasted_iota(jnp.int32, (8 * OC, 8 * OC), 0)
    b_idx = lax.broadcasted_iota(jnp.int32, (8 * OC, 8 * OC), 1)
    return jnp.where(a_idx // OC == b_idx // OC, 1.0, 0.0).astype(jnp.float32)


def _prep_body(x_ref, wt_ref, b_ref, o_ref):
    y = jnp.dot(x_ref[...], wt_ref[...], preferred_element_type=jnp.float32)
    y = jnp.maximum(y + b_ref[...], 0.0)
    sel = _selector()
    sq = jnp.dot(y * y, sel, preferred_element_type=jnp.float32)
    sqb = jnp.dot(sq, sel.T, preferred_element_type=jnp.float32)
    o_ref[...] = y / jnp.maximum(jnp.sqrt(sqb), 1e-12)


def _route_body(x_ref, n_ref, acc_ref, o_ref):
    del acc_ref
    R = M // 8
    W8 = 8 * D
    xb = x_ref[...]
    nb8 = n_ref[...]
    sel = _selector()
    sel8 = _selector8()
    j8 = _blocksum8()
    u = xb
    for it in range(ROUT_IT):
        u8 = jnp.concatenate([u] * 8, axis=1)
        ue8 = jnp.broadcast_to(u8[:, None, :], (BN, R, W8)).reshape(BN * R, W8)
        p8 = jnp.dot(nb8 * ue8, sel8, preferred_element_type=jnp.float32)
        e8 = jnp.exp(p8)
        s8 = jnp.dot(e8, j8, preferred_element_type=jnp.float32)
        pn8 = e8 / s8
        pe8 = jnp.dot(pn8, sel8.T, preferred_element_type=jnp.float32)
        un8 = jnp.sum((pe8 * nb8).reshape(BN, R, W8), axis=1)
        t4 = un8[:, : W8 // 2] + un8[:, W8 // 2:]
        t2 = t4[:, : W8 // 4] + t4[:, W8 // 4:]
        u = t2[:, :D] + t2[:, D:] + xb
        if it < ROUT_IT - 1:
            sq = jnp.dot(u * u, sel, preferred_element_type=jnp.float32)
            sqb = jnp.dot(sq, sel.T, preferred_element_type=jnp.float32)
            u = u / jnp.maximum(jnp.sqrt(sqb), 1e-12)
    o_ref[...] = u


def _tc_prep(x, wt, b2):
    return pl.pallas_call(
        _prep_body,
        grid=(N // BN,),
        in_specs=[
            pl.BlockSpec((BN, IN_D), lambda i: (i, 0)),
            pl.BlockSpec((IN_D, D), lambda i: (0, 0)),
            pl.BlockSpec((1, D), lambda i: (0, 0)),
        ],
        out_specs=pl.BlockSpec((BN, D), lambda i: (i, 0)),
        out_shape=jax.ShapeDtypeStruct((N, D), jnp.float32),
    )(x, wt, b2)


def _sc_gather_slice(table, neighbor_id, s):
    edge_base = s * EDGES_SL
    mesh = plsc.VectorSubcoreMesh(
        core_axis_name="c", subcore_axis_name="s",
        num_cores=NC, num_subcores=NS)

    @functools.partial(
        pl.kernel,
        out_type=jax.ShapeDtypeStruct((EDGES_SL, D), jnp.float32),
        mesh=mesh,
        scratch_types=[
            pltpu.VMEM((CHUNK,), jnp.int32),
            pltpu.VMEM((CHUNK, D), jnp.float32),
            pltpu.SemaphoreType.DMA,
        ],
        compiler_params=pltpu.CompilerParams(use_tc_tiling_on_sc=False),
    )
    def gather_k(table_hbm, idx_hbm, out_hbm, idx_v, rows_v, sem):
        wid = lax.axis_index("s") * NC + lax.axis_index("c")
        base_w = wid * PER_W

        def body(t, carry):
            base = base_w + t * CHUNK
            pltpu.sync_copy(idx_hbm.at[pl.ds(edge_base + base, CHUNK)], idx_v)
            pltpu.async_copy(table_hbm.at[idx_v], rows_v, sem).wait()
            pltpu.sync_copy(rows_v, out_hbm.at[pl.ds(base, CHUNK)])
            return carry

        lax.fori_loop(0, N_CHUNKS, body, 0)

    return gather_k(table, neighbor_id)


def _tc_route_slice(table, neighbors2, acc, s):
    blk_off = s * (NODES_SL // BN)
    return pl.pallas_call(
        _route_body,
        grid=(NODES_SL // BN,),
        in_specs=[
            pl.BlockSpec((BN, D), lambda i: (i + blk_off, 0)),
            pl.BlockSpec((BN * M // 8, 8 * D), lambda i: (i, 0)),
            pl.BlockSpec((BN, D), lambda i: (i + blk_off, 0)),
        ],
        out_specs=pl.BlockSpec((BN, D), lambda i: (i + blk_off, 0)),
        out_shape=jax.ShapeDtypeStruct((N, D), jnp.float32),
        input_output_aliases={2: 0},
    )(table, neighbors2, acc)


def kernel(x, neighbor_id, W, b):
    wt = W.T
    b2 = b.reshape(1, D)
    table = _tc_prep(x, wt, b2)
    acc = jnp.zeros((N, D), dtype=jnp.float32)
    for s in range(N_SLICES):
        flat = _sc_gather_slice(table, neighbor_id, s)
        nb8 = flat.reshape(EDGES_SL // 8, 8 * D)
        acc = _tc_route_slice(table, nb8, acc, s)
    return acc


# speedup vs baseline: 2.7020x; 2.7020x over previous
import functools

import jax
import jax.numpy as jnp
from jax import lax
from jax.experimental import pallas as pl
from jax.experimental.pallas import tpu as pltpu
from jax.experimental.pallas import tpu_sc as plsc

N = 50000
M = 16
IN_D = 128
OC = 8
KD = 8
D = OC * KD
ROUT_IT = 2

N_SLICES = 5
NODES_SL = N // N_SLICES
EDGES_SL = NODES_SL * M

BN = 1000
NC, NS = 2, 16
NW = NC * NS
PER_W = EDGES_SL // NW
CHUNK = 1000
N_CHUNKS = PER_W // CHUNK


def _selector():
    d_idx = lax.broadcasted_iota(jnp.int32, (D, OC), 0)
    c_idx = lax.broadcasted_iota(jnp.int32, (D, OC), 1)
    return jnp.where(d_idx // KD == c_idx, 1.0, 0.0).astype(jnp.float32)


def _selector2():
    d_idx = lax.broadcasted_iota(jnp.int32, (2 * D, 2 * OC), 0)
    c_idx = lax.broadcasted_iota(jnp.int32, (2 * D, 2 * OC), 1)
    return jnp.where(d_idx // KD == c_idx, 1.0, 0.0).astype(jnp.float32)


def _blocksum2():
    a_idx = lax.broadcasted_iota(jnp.int32, (2 * OC, 2 * OC), 0)
    b_idx = lax.broadcasted_iota(jnp.int32, (2 * OC, 2 * OC), 1)
    return jnp.where(a_idx // OC == b_idx // OC, 1.0, 0.0).astype(jnp.float32)


def _prep_body(x_ref, wt_ref, b_ref, o_ref):
    y = jnp.dot(x_ref[...], wt_ref[...], preferred_element_type=jnp.float32)
    y = jnp.maximum(y + b_ref[...], 0.0)
    sel = _selector()
    sq = jnp.dot(y * y, sel, preferred_element_type=jnp.float32)
    sqb = jnp.dot(sq, sel.T, preferred_element_type=jnp.float32)
    o_ref[...] = y / jnp.maximum(jnp.sqrt(sqb), 1e-12)


def _route_body(x_ref, n_ref, acc_ref, o_ref):
    del acc_ref
    MH = M // 2
    xb = x_ref[...]
    nb2 = n_ref[...]
    sel = _selector()
    sel2 = _selector2()
    j2 = _blocksum2()
    u = xb
    for it in range(ROUT_IT):
        u2 = jnp.concatenate([u, u], axis=1)
        ue2 = jnp.broadcast_to(u2[:, None, :], (BN, MH, 2 * D)).reshape(BN * MH, 2 * D)
        p2 = jnp.dot(nb2 * ue2, sel2, preferred_element_type=jnp.float32)
        e2 = jnp.exp(p2)
        s2 = jnp.dot(e2, j2, preferred_element_type=jnp.float32)
        pn2 = e2 / s2
        pe2 = jnp.dot(pn2, sel2.T, preferred_element_type=jnp.float32)
        un2 = jnp.sum((pe2 * nb2).reshape(BN, MH, 2 * D), axis=1)
        u = un2[:, :D] + un2[:, D:] + xb
        if it < ROUT_IT - 1:
            sq = jnp.dot(u * u, sel, preferred_element_type=jnp.float32)
            sqb = jnp.dot(sq, sel.T, preferred_element_type=jnp.float32)
            u = u / jnp.maximum(jnp.sqrt(sqb), 1e-12)
    o_ref[...] = u


def _tc_prep(x, wt, b2):
    return pl.pallas_call(
        _prep_body,
        grid=(N // BN,),
        in_specs=[
            pl.BlockSpec((BN, IN_D), lambda i: (i, 0)),
            pl.BlockSpec((IN_D, D), lambda i: (0, 0)),
            pl.BlockSpec((1, D), lambda i: (0, 0)),
        ],
        out_specs=pl.BlockSpec((BN, D), lambda i: (i, 0)),
        out_shape=jax.ShapeDtypeStruct((N, D), jnp.float32),
    )(x, wt, b2)


def _sc_gather_slice(table, neighbor_id, s):
    edge_base = s * EDGES_SL
    mesh = plsc.VectorSubcoreMesh(
        core_axis_name="c", subcore_axis_name="s",
        num_cores=NC, num_subcores=NS)

    @functools.partial(
        pl.kernel,
        out_type=jax.ShapeDtypeStruct((EDGES_SL, D), jnp.float32),
        mesh=mesh,
        scratch_types=[
            pltpu.VMEM((CHUNK,), jnp.int32),
            pltpu.VMEM((CHUNK, D), jnp.float32),
            pltpu.SemaphoreType.DMA,
        ],
        compiler_params=pltpu.CompilerParams(use_tc_tiling_on_sc=False),
    )
    def gather_k(table_hbm, idx_hbm, out_hbm, idx_v, rows_v, sem):
        wid = lax.axis_index("s") * NC + lax.axis_index("c")
        base_w = wid * PER_W

        def body(t, carry):
            base = base_w + t * CHUNK
            pltpu.sync_copy(idx_hbm.at[pl.ds(edge_base + base, CHUNK)], idx_v)
            pltpu.async_copy(table_hbm.at[idx_v], rows_v, sem).wait()
            pltpu.sync_copy(rows_v, out_hbm.at[pl.ds(base, CHUNK)])
            return carry

        lax.fori_loop(0, N_CHUNKS, body, 0)

    return gather_k(table, neighbor_id)


def _tc_route_slice(table, neighbors2, acc, s):
    blk_off = s * (NODES_SL // BN)
    return pl.pallas_call(
        _route_body,
        grid=(NODES_SL // BN,),
        in_specs=[
            pl.BlockSpec((BN, D), lambda i: (i + blk_off, 0)),
            pl.BlockSpec((BN * M // 2, 2 * D), lambda i: (i, 0)),
            pl.BlockSpec((BN, D), lambda i: (i + blk_off, 0)),
        ],
        out_specs=pl.BlockSpec((BN, D), lambda i: (i + blk_off, 0)),
        out_shape=jax.ShapeDtypeStruct((N, D), jnp.float32),
        input_output_aliases={2: 0},
    )(table, neighbors2, acc)


def kernel(x, neighbor_id, W, b):
    wt = W.T
    b2 = b.reshape(1, D)
    table = _tc_prep(x, wt, b2)
    acc = jnp.zeros((N, D), dtype=jnp.float32)
    for s in range(N_SLICES):
        flat = _sc_gather_slice(table, neighbor_id, s)
        nb2 = flat.reshape(EDGES_SL // 2, 2 * D)
        acc = _tc_route_slice(table, nb2, acc, s)
    return acc


# speedup vs baseline: 2.7810x; 1.0292x over previous
import functools

import jax
import jax.numpy as jnp
from jax import lax
from jax.experimental import pallas as pl
from jax.experimental.pallas import tpu as pltpu
from jax.experimental.pallas import tpu_sc as plsc

N = 50000
M = 16
IN_D = 128
OC = 8
KD = 8
D = OC * KD
ROUT_IT = 2

N_SLICES = 5
NODES_SL = N // N_SLICES
EDGES_SL = NODES_SL * M

BN = 1000
NC, NS = 2, 16
NW = NC * NS
PER_W = EDGES_SL // NW
CHUNK = 1000
N_CHUNKS = PER_W // CHUNK


def _selector():
    d_idx = lax.broadcasted_iota(jnp.int32, (D, OC), 0)
    c_idx = lax.broadcasted_iota(jnp.int32, (D, OC), 1)
    return jnp.where(d_idx // KD == c_idx, 1.0, 0.0).astype(jnp.float32)


def _selector2():
    d_idx = lax.broadcasted_iota(jnp.int32, (2 * D, 2 * OC), 0)
    c_idx = lax.broadcasted_iota(jnp.int32, (2 * D, 2 * OC), 1)
    return jnp.where(d_idx // KD == c_idx, 1.0, 0.0).astype(jnp.float32)


def _blocksum2():
    a_idx = lax.broadcasted_iota(jnp.int32, (2 * OC, 2 * OC), 0)
    b_idx = lax.broadcasted_iota(jnp.int32, (2 * OC, 2 * OC), 1)
    return jnp.where(a_idx // OC == b_idx // OC, 1.0, 0.0).astype(jnp.float32)


def _prep_body(x_ref, wt_ref, b_ref, o_ref):
    y = jnp.dot(x_ref[...], wt_ref[...], preferred_element_type=jnp.float32)
    y = jnp.maximum(y + b_ref[...], 0.0)
    sel = _selector()
    sq = jnp.dot(y * y, sel, preferred_element_type=jnp.float32)
    sqb = jnp.dot(sq, sel.T, preferred_element_type=jnp.float32)
    o_ref[...] = y * lax.rsqrt(jnp.maximum(sqb, 1e-24))


def _route_body(x_ref, n_ref, acc_ref, o_ref):
    del acc_ref
    MH = M // 2
    xb = x_ref[...]
    nb2 = n_ref[...]
    sel = _selector()
    sel2 = _selector2()
    j2 = _blocksum2()
    u = xb
    for it in range(ROUT_IT):
        u2 = jnp.concatenate([u, u], axis=1)
        ue2 = jnp.broadcast_to(u2[:, None, :], (BN, MH, 2 * D)).reshape(BN * MH, 2 * D)
        p2 = jnp.dot(nb2 * ue2, sel2, preferred_element_type=jnp.float32)
        e2 = jnp.exp(p2)
        s2 = jnp.dot(e2, j2, preferred_element_type=jnp.float32)
        pn2 = e2 / s2
        pe2 = jnp.dot(pn2, sel2.T, preferred_element_type=jnp.float32)
        un2 = jnp.sum((pe2 * nb2).reshape(BN, MH, 2 * D), axis=1)
        u = un2[:, :D] + un2[:, D:] + xb
        if it < ROUT_IT - 1:
            sq = jnp.dot(u * u, sel, preferred_element_type=jnp.float32)
            sqb = jnp.dot(sq, sel.T, preferred_element_type=jnp.float32)
            u = u * lax.rsqrt(jnp.maximum(sqb, 1e-24))
    o_ref[...] = u


def _tc_prep(x, wt, b2):
    return pl.pallas_call(
        _prep_body,
        grid=(N // BN,),
        in_specs=[
            pl.BlockSpec((BN, IN_D), lambda i: (i, 0)),
            pl.BlockSpec((IN_D, D), lambda i: (0, 0)),
            pl.BlockSpec((1, D), lambda i: (0, 0)),
        ],
        out_specs=pl.BlockSpec((BN, D), lambda i: (i, 0)),
        out_shape=jax.ShapeDtypeStruct((N, D), jnp.float32),
    )(x, wt, b2)


def _sc_gather_slice(table, neighbor_id, s):
    edge_base = s * EDGES_SL
    mesh = plsc.VectorSubcoreMesh(
        core_axis_name="c", subcore_axis_name="s",
        num_cores=NC, num_subcores=NS)

    @functools.partial(
        pl.kernel,
        out_type=jax.ShapeDtypeStruct((EDGES_SL, D), jnp.float32),
        mesh=mesh,
        scratch_types=[
            pltpu.VMEM((CHUNK,), jnp.int32),
            pltpu.VMEM((CHUNK, D), jnp.float32),
            pltpu.SemaphoreType.DMA,
        ],
        compiler_params=pltpu.CompilerParams(use_tc_tiling_on_sc=False),
    )
    def gather_k(table_hbm, idx_hbm, out_hbm, idx_v, rows_v, sem):
        wid = lax.axis_index("s") * NC + lax.axis_index("c")
        base_w = wid * PER_W

        def body(t, carry):
            base = base_w + t * CHUNK
            pltpu.sync_copy(idx_hbm.at[pl.ds(edge_base + base, CHUNK)], idx_v)
            pltpu.async_copy(table_hbm.at[idx_v], rows_v, sem).wait()
            pltpu.sync_copy(rows_v, out_hbm.at[pl.ds(base, CHUNK)])
            return carry

        lax.fori_loop(0, N_CHUNKS, body, 0)

    return gather_k(table, neighbor_id)


def _tc_route_slice(table, neighbors2, acc, s):
    blk_off = s * (NODES_SL // BN)
    return pl.pallas_call(
        _route_body,
        grid=(NODES_SL // BN,),
        in_specs=[
            pl.BlockSpec((BN, D), lambda i: (i + blk_off, 0)),
            pl.BlockSpec((BN * M // 2, 2 * D), lambda i: (i, 0)),
            pl.BlockSpec((BN, D), lambda i: (i + blk_off, 0)),
        ],
        out_specs=pl.BlockSpec((BN, D), lambda i: (i + blk_off, 0)),
        out_shape=jax.ShapeDtypeStruct((N, D), jnp.float32),
        input_output_aliases={2: 0},
    )(table, neighbors2, acc)


def kernel(x, neighbor_id, W, b):
    wt = W.T
    b2 = b.reshape(1, D)
    table = _tc_prep(x, wt, b2)
    acc = jnp.zeros((N, D), dtype=jnp.float32)
    for s in range(N_SLICES):
        flat = _sc_gather_slice(table, neighbor_id, s)
        nb2 = flat.reshape(EDGES_SL // 2, 2 * D)
        acc = _tc_route_slice(table, nb2, acc, s)
    return acc
